# Initial kernel scaffold; baseline (speedup 1.0000x reference)
import jax
import jax.numpy as jnp
from jax.experimental import pallas as pl


def kernel(x, table):
    raise NotImplementedError("write your pallas kernel here")


# speedup vs baseline: 1.0946x; 1.0946x over previous
import functools

import jax
import jax.numpy as jnp
from jax import lax
from jax.experimental import pallas as pl
from jax.experimental.pallas import tpu as pltpu
from jax.experimental.pallas import tpu_sc as plsc

_NUM_CORES = 2
_NUM_SUBCORES = 16
_NUM_WORKERS = _NUM_CORES * _NUM_SUBCORES


@functools.partial(jax.jit, static_argnames=("chunk",))
def _gather_rows(idx, table, chunk=1024):
    (B,) = idx.shape
    _, D = table.shape
    b_per_w = B // _NUM_WORKERS
    n_chunks = b_per_w // chunk
    mesh = plsc.VectorSubcoreMesh(core_axis_name="c", subcore_axis_name="s")

    @functools.partial(
        pl.kernel,
        mesh=mesh,
        out_type=jax.ShapeDtypeStruct((B, D), jnp.float32),
        scratch_types=[
            pltpu.VMEM((chunk,), jnp.int32),
            pltpu.VMEM((chunk, D), jnp.float32),
            pltpu.SemaphoreType.DMA,
        ],
        compiler_params=pltpu.CompilerParams(use_tc_tiling_on_sc=False),
    )
    def k(idx_hbm, table_hbm, out_hbm, idx_v, rows_v, sem):
        wid = lax.axis_index("s") * _NUM_CORES + lax.axis_index("c")
        base = wid * b_per_w

        def body(i, carry):
            off = base + i * chunk
            pltpu.sync_copy(idx_hbm.at[pl.ds(off, chunk)], idx_v)
            pltpu.async_copy(table_hbm.at[idx_v], rows_v, sem).wait()
            pltpu.sync_copy(rows_v, out_hbm.at[pl.ds(off, chunk)])
            return carry

        lax.fori_loop(0, n_chunks, body, 0)

    return k(idx, table)


def kernel(x, table):
    bt, hist = x.shape
    _, d = table.shape
    idx = x.reshape(-1).astype(jnp.int32)
    out = _gather_rows(idx, table)
    return out.reshape(bt, hist, d)


# speedup vs baseline: 1.1095x; 1.0136x over previous
import functools

import jax
import jax.numpy as jnp
from jax import lax
from jax.experimental import pallas as pl
from jax.experimental.pallas import tpu as pltpu
from jax.experimental.pallas import tpu_sc as plsc

_NUM_CORES = 2
_NUM_SUBCORES = 16
_NUM_WORKERS = _NUM_CORES * _NUM_SUBCORES


@functools.partial(jax.jit, static_argnames=("chunk",))
def _gather_rows(idx, table, chunk=1600):
    (B,) = idx.shape
    _, D = table.shape
    b_per_w = B // _NUM_WORKERS
    n_chunks = b_per_w // chunk
    assert n_chunks % 2 == 0 and n_chunks >= 4
    n_outer = n_chunks // 2
    mesh = plsc.VectorSubcoreMesh(core_axis_name="c", subcore_axis_name="s")

    @functools.partial(
        pl.kernel,
        mesh=mesh,
        out_type=jax.ShapeDtypeStruct((B, D), jnp.float32),
        scratch_types=[
            pltpu.VMEM((chunk,), jnp.int32),
            pltpu.VMEM((chunk,), jnp.int32),
            pltpu.VMEM((chunk, D), jnp.float32),
            pltpu.VMEM((chunk, D), jnp.float32),
            pltpu.SemaphoreType.DMA,
            pltpu.SemaphoreType.DMA,
            pltpu.SemaphoreType.DMA,
            pltpu.SemaphoreType.DMA,
            pltpu.SemaphoreType.DMA,
            pltpu.SemaphoreType.DMA,
        ],
        compiler_params=pltpu.CompilerParams(use_tc_tiling_on_sc=False),
    )
    def k(idx_hbm, table_hbm, out_hbm, idx0, idx1, rows0, rows1,
          li0, li1, g0, g1, s0, s1):
        wid = lax.axis_index("s") * _NUM_CORES + lax.axis_index("c")
        base = wid * b_per_w
        idx_v, rows_v = (idx0, idx1), (rows0, rows1)
        sem_li, sem_g, sem_s = (li0, li1), (g0, g1), (s0, s1)

        def li_copy(i, b):
            return pltpu.make_async_copy(
                idx_hbm.at[pl.ds(base + i * chunk, chunk)], idx_v[b], sem_li[b])

        def g_copy(b):
            return pltpu.make_async_copy(
                table_hbm.at[idx_v[b]], rows_v[b], sem_g[b])

        def s_copy(i, b):
            return pltpu.make_async_copy(
                rows_v[b], out_hbm.at[pl.ds(base + i * chunk, chunk)], sem_s[b])

        for b in range(2):
            li_copy(b, b).start()
        for b in range(2):
            li_copy(b, b).wait()
            g_copy(b).start()
            g_copy(b).wait()
            li_copy(2 + b, b).start()
            s_copy(b, b).start()

        def body(g, carry):
            for b in range(2):
                i = 2 * g + b
                li_copy(i, b).wait()
                s_copy(i - 2, b).wait()
                g_copy(b).start()
                g_copy(b).wait()
                li_copy(i + 2, b).start()
                s_copy(i, b).start()
            return carry

        lax.fori_loop(1, n_outer - 1, body, 0)

        for b in range(2):
            i = n_chunks - 2 + b
            li_copy(i, b).wait()
            s_copy(i - 2, b).wait()
            g_copy(b).start()
            g_copy(b).wait()
            s_copy(i, b).start()
        for b in range(2):
            s_copy(n_chunks - 2 + b, b).wait()

    return k(idx, table)


def kernel(x, table):
    bt, hist = x.shape
    _, d = table.shape
    idx = x.reshape(-1).astype(jnp.int32)
    out = _gather_rows(idx, table)
    return out.reshape(bt, hist, d)


# speedup vs baseline: 1.6491x; 1.4864x over previous
import functools

import jax
import jax.numpy as jnp
from jax import lax
from jax.experimental import pallas as pl
from jax.experimental.pallas import tpu as pltpu
from jax.experimental.pallas import tpu_sc as plsc

_NUM_CORES = 2
_NUM_SUBCORES = 16
_NUM_WORKERS = _NUM_CORES * _NUM_SUBCORES

_BT = 16384
_H = 50
_D = 32
_BPW = _BT // _NUM_WORKERS
_TCW = _BPW // 128


@jax.jit
def _gather_embed(idx_hm, table):
    mesh = plsc.VectorSubcoreMesh(core_axis_name="c", subcore_axis_name="s")

    @functools.partial(
        pl.kernel,
        mesh=mesh,
        out_type=jax.ShapeDtypeStruct((_H, _D // 8, _BT // 128, 8, 128),
                                      jnp.float32),
        scratch_types=[
            pltpu.VMEM((_BPW,), jnp.int32),
            pltpu.VMEM((_BPW,), jnp.int32),
            pltpu.VMEM((_BPW, _D), jnp.float32),
            pltpu.VMEM((_BPW, _D), jnp.float32),
            pltpu.VMEM((_D // 8, _TCW, 8, 128), jnp.float32),
            pltpu.VMEM((_D // 8, _TCW, 8, 128), jnp.float32),
            pltpu.SemaphoreType.DMA,
            pltpu.SemaphoreType.DMA,
            pltpu.SemaphoreType.DMA,
            pltpu.SemaphoreType.DMA,
            pltpu.SemaphoreType.DMA,
            pltpu.SemaphoreType.DMA,
        ],
        compiler_params=pltpu.CompilerParams(
            use_tc_tiling_on_sc=False, needs_layout_passes=False),
    )
    def k(idx_hbm, table_hbm, out_hbm, idx0, idx1, rows0, rows1, t0, t1,
          li0, li1, g0, g1, s0, s1):
        wid = lax.axis_index("s") * _NUM_CORES + lax.axis_index("c")
        b0 = wid * _BPW
        tc0 = wid * _TCW
        idx_v, rows_v, t_v = (idx0, idx1), (rows0, rows1), (t0, t1)
        sem_li, sem_g, sem_s = (li0, li1), (g0, g1), (s0, s1)
        iota16 = lax.iota(jnp.int32, 16)
        zeros16 = jnp.zeros((16,), jnp.int32)

        def li_copy(h, b):
            return pltpu.make_async_copy(
                idx_hbm.at[pl.ds(h * _BT + b0, _BPW)], idx_v[b], sem_li[b])

        def g_copy(b):
            return pltpu.make_async_copy(
                table_hbm.at[idx_v[b]], rows_v[b], sem_g[b])

        def s_copy(h, b):
            return pltpu.make_async_copy(
                t_v[b], out_hbm.at[h, :, pl.ds(tc0, _TCW), :, :], sem_s[b])

        def transpose(b):
            rows, t = rows_v[b], t_v[b]

            def jg_body(jg, carry):
                tcl = jg // 8
                l0 = (jg % 8) * 16
                row0 = jg * 16
                for c in range(_D):
                    v = plsc.load_gather(rows, [row0 + iota16, c + zeros16])
                    t[c // 8, tcl, c % 8, pl.ds(l0, 16)] = v
                return carry

            lax.fori_loop(0, _BPW // 16, jg_body, 0)

        li_copy(0, 0).start()
        li_copy(1, 1).start()
        li_copy(0, 0).wait()
        g_copy(0).start()
        for b, h in ((0, 0), (1, 1)):
            nb = 1 - b
            g_copy(b).wait()
            li_copy(h + 1, nb).wait()
            g_copy(nb).start()
            li_copy(h + 2, b).start()
            transpose(b)
            s_copy(h, b).start()

        def body(g_i, carry):
            for b in range(2):
                h = 2 * g_i + b
                nb = 1 - b
                g_copy(b).wait()
                li_copy(h + 1, nb).wait()
                g_copy(nb).start()
                li_copy(h + 2, b).start()
                s_copy(h - 2, b).wait()
                transpose(b)
                s_copy(h, b).start()
            return carry

        lax.fori_loop(1, (_H - 2) // 2, body, 0)

        g_copy(0).wait()
        li_copy(_H - 1, 1).wait()
        g_copy(1).start()
        s_copy(_H - 4, 0).wait()
        transpose(0)
        s_copy(_H - 2, 0).start()

        g_copy(1).wait()
        s_copy(_H - 3, 1).wait()
        transpose(1)
        s_copy(_H - 1, 1).start()

        s_copy(_H - 2, 0).wait()
        s_copy(_H - 1, 1).wait()

    return k(idx_hm, table)


def kernel(x, table):
    idx_hm = x.T.reshape(-1).astype(jnp.int32)
    out6 = _gather_embed(idx_hm, table)
    return out6.transpose(2, 4, 0, 1, 3).reshape(_BT, _H, _D)


# speedup vs baseline: 1.6515x; 1.0015x over previous
import functools

import jax
import jax.numpy as jnp
from jax import lax
from jax.experimental import pallas as pl
from jax.experimental.pallas import tpu as pltpu
from jax.experimental.pallas import tpu_sc as plsc

_NUM_CORES = 2
_NUM_SUBCORES = 16
_NUM_WORKERS = _NUM_CORES * _NUM_SUBCORES

_BT = 16384
_H = 50
_D = 32
_BPW = _BT // _NUM_WORKERS
_TCW = _BPW // 128


@jax.jit
def _gather_embed(idx_hm, table):
    mesh = plsc.VectorSubcoreMesh(core_axis_name="c", subcore_axis_name="s")

    @functools.partial(
        pl.kernel,
        mesh=mesh,
        out_type=jax.ShapeDtypeStruct((_H, _D // 8, (_BT // 128) * 1024),
                                      jnp.float32),
        scratch_types=[
            pltpu.VMEM((_BPW,), jnp.int32),
            pltpu.VMEM((_BPW,), jnp.int32),
            pltpu.VMEM((_BPW, _D), jnp.float32),
            pltpu.VMEM((_BPW, _D), jnp.float32),
            pltpu.VMEM((_D // 8, _TCW * 1024), jnp.float32),
            pltpu.VMEM((_D // 8, _TCW * 1024), jnp.float32),
            pltpu.SemaphoreType.DMA,
            pltpu.SemaphoreType.DMA,
            pltpu.SemaphoreType.DMA,
            pltpu.SemaphoreType.DMA,
            pltpu.SemaphoreType.DMA,
            pltpu.SemaphoreType.DMA,
        ],
        compiler_params=pltpu.CompilerParams(
            use_tc_tiling_on_sc=False, needs_layout_passes=False),
    )
    def k(idx_hbm, table_hbm, out_hbm, idx0, idx1, rows0, rows1, t0, t1,
          li0, li1, g0, g1, s0, s1):
        wid = lax.axis_index("s") * _NUM_CORES + lax.axis_index("c")
        b0 = wid * _BPW
        tc0 = wid * _TCW
        idx_v, rows_v, t_v = (idx0, idx1), (rows0, rows1), (t0, t1)
        sem_li, sem_g, sem_s = (li0, li1), (g0, g1), (s0, s1)
        iota16 = lax.iota(jnp.int32, 16)
        zeros16 = jnp.zeros((16,), jnp.int32)
        vcols = [c + zeros16 for c in range(_D)]
        cpart = [(c % 8) * 128 for c in range(_D)]

        def li_copy(h, b):
            return pltpu.make_async_copy(
                idx_hbm.at[pl.ds(h * _BT + b0, _BPW)], idx_v[b], sem_li[b])

        def g_copy(b):
            return pltpu.make_async_copy(
                table_hbm.at[idx_v[b]], rows_v[b], sem_g[b])

        def s_copy(h, b):
            return pltpu.make_async_copy(
                t_v[b], out_hbm.at[h, :, pl.ds(tc0 * 1024, _TCW * 1024)],
                sem_s[b])

        def transpose(b):
            rows, t = rows_v[b], t_v[b]

            def jg_body(jg, carry):
                dbase = (jg // 8) * 1024 + (jg % 8) * 16
                vrow = jg * 16 + iota16
                for c in range(_D):
                    v = plsc.load_gather(rows, [vrow, vcols[c]])
                    t[c // 8, pl.ds(dbase + cpart[c], 16)] = v
                return carry

            lax.fori_loop(0, _BPW // 16, jg_body, 0)

        li_copy(0, 0).start()
        li_copy(1, 1).start()
        li_copy(0, 0).wait()
        g_copy(0).start()
        for b, h in ((0, 0), (1, 1)):
            nb = 1 - b
            g_copy(b).wait()
            li_copy(h + 1, nb).wait()
            g_copy(nb).start()
            li_copy(h + 2, b).start()
            transpose(b)
            s_copy(h, b).start()

        def body(g_i, carry):
            for b in range(2):
                h = 2 * g_i + b
                nb = 1 - b
                g_copy(b).wait()
                li_copy(h + 1, nb).wait()
                g_copy(nb).start()
                li_copy(h + 2, b).start()
                s_copy(h - 2, b).wait()
                transpose(b)
                s_copy(h, b).start()
            return carry

        lax.fori_loop(1, (_H - 2) // 2, body, 0)

        g_copy(0).wait()
        li_copy(_H - 1, 1).wait()
        g_copy(1).start()
        s_copy(_H - 4, 0).wait()
        transpose(0)
        s_copy(_H - 2, 0).start()

        g_copy(1).wait()
        s_copy(_H - 3, 1).wait()
        transpose(1)
        s_copy(_H - 1, 1).start()

        s_copy(_H - 2, 0).wait()
        s_copy(_H - 1, 1).wait()

    return k(idx_hm, table)


def kernel(x, table):
    idx_hm = x.T.reshape(-1).astype(jnp.int32)
    out6 = _gather_embed(idx_hm, table)
    out6 = out6.reshape(_H, _D // 8, _BT // 128, 8, 128)
    return out6.transpose(2, 4, 0, 1, 3).reshape(_BT, _H, _D)
